# baseline (device time: 14037 ns/iter reference)
import jax
import jax.numpy as jnp
from jax import lax
from jax.experimental import pallas as pl
from jax.experimental.pallas import tpu as pltpu

N_DEV = 8
M_ROWS = 2048
CHUNK = 128
N_CHUNK = M_ROWS // CHUNK
G = 4
CPG = N_CHUNK // G


def kernel(x):
    m_rows, n_loc = x.shape

    def body(x_ref, out_ref, s_ref, comm_ref, send_sems, recv_sems):
        my = lax.axis_index("i")

        barrier_sem = pltpu.get_barrier_semaphore()
        for d in range(1, N_DEV):
            pl.semaphore_signal(
                barrier_sem, inc=1,
                device_id=((my + d) % N_DEV,),
                device_id_type=pl.DeviceIdType.MESH,
            )

        WAIT_AT = 1

        def send_group(g):
            out = []
            for d in range(1, N_DEV):
                rdma = pltpu.make_async_remote_copy(
                    src_ref=s_ref.at[pl.ds(g * CPG, CPG)],
                    dst_ref=comm_ref.at[d - 1, pl.ds(g * CPG, CPG)],
                    send_sem=send_sems.at[g, d - 1],
                    recv_sem=recv_sems.at[g, d - 1],
                    device_id=((my + d) % N_DEV,),
                    device_id_type=pl.DeviceIdType.MESH,
                )
                rdma.start()
                out.append(rdma)
            return out

        rdmas = []
        for g in range(G):
            for j in range(CPG):
                i = g * CPG + j
                xb = x_ref[pl.ds(i * CHUNK, CHUNK), :].reshape(1, CHUNK, n_loc)
                s_ref[pl.ds(i, 1), :] = jnp.sum(jnp.exp(xb), axis=2)
            if g == WAIT_AT:
                pl.semaphore_wait(barrier_sem, N_DEV - 1)
                for gp in range(WAIT_AT + 1):
                    rdmas += send_group(gp)
            elif g > WAIT_AT:
                rdmas += send_group(g)

        k = 0
        for g in range(G):
            for _ in range(N_DEV - 1):
                rdmas[k].wait()
                k += 1
            s_tot = s_ref[pl.ds(g * CPG, CPG), :] + jnp.sum(
                comm_ref[:, pl.ds(g * CPG, CPG), :], axis=0
            )
            ln_s = jnp.log(s_tot)
            for j in range(CPG):
                i = g * CPG + j
                xb = x_ref[pl.ds(i * CHUNK, CHUNK), :].reshape(1, CHUNK, n_loc)
                ob = jnp.exp(xb - ln_s[j : j + 1, :][:, :, None])
                out_ref[pl.ds(i * CHUNK, CHUNK), :] = ob.reshape(
                    CHUNK, n_loc
                ).astype(jnp.bfloat16)

    return pl.pallas_call(
        body,
        out_shape=jax.ShapeDtypeStruct((m_rows, n_loc), jnp.bfloat16),
        in_specs=[pl.BlockSpec(memory_space=pltpu.VMEM)],
        out_specs=pl.BlockSpec(memory_space=pltpu.VMEM),
        scratch_shapes=[
            pltpu.VMEM((N_CHUNK, CHUNK), jnp.float32),
            pltpu.VMEM((N_DEV - 1, N_CHUNK, CHUNK), jnp.float32),
            pltpu.SemaphoreType.DMA((G, N_DEV - 1)),
            pltpu.SemaphoreType.DMA((G, N_DEV - 1)),
        ],
        compiler_params=pltpu.CompilerParams(collective_id=0),
    )(x)


# device time: 13700 ns/iter; 1.0246x vs baseline; 1.0246x over previous
import jax
import jax.numpy as jnp
from jax import lax
from jax.experimental import pallas as pl
from jax.experimental.pallas import tpu as pltpu

N_DEV = 8
M_ROWS = 2048
CHUNK = 128
N_CHUNK = M_ROWS // CHUNK
G = 4
CPG = N_CHUNK // G


def kernel(x):
    m_rows, n_loc = x.shape

    def body(x_ref, out_ref, e_ref, s_ref, comm_ref, send_sems, recv_sems):
        my = lax.axis_index("i")

        barrier_sem = pltpu.get_barrier_semaphore()
        for d in range(1, N_DEV):
            pl.semaphore_signal(
                barrier_sem, inc=1,
                device_id=((my + d) % N_DEV,),
                device_id_type=pl.DeviceIdType.MESH,
            )

        ones_row = jnp.ones((1, n_loc), jnp.bfloat16)

        rdmas = []
        for g in range(G):
            for j in range(CPG):
                i = g * CPG + j
                xb = x_ref[pl.ds(i * CHUNK, CHUNK), :]
                eb = jnp.exp(xb).astype(jnp.bfloat16)
                e_ref[pl.ds(i * CHUNK, CHUNK), :] = eb
                s_ref[pl.ds(i, 1), :] = lax.dot_general(
                    ones_row, eb,
                    (((1,), (1,)), ((), ())),
                    preferred_element_type=jnp.float32,
                )
            if g == 0:
                pl.semaphore_wait(barrier_sem, N_DEV - 1)
            for d in range(1, N_DEV):
                rdma = pltpu.make_async_remote_copy(
                    src_ref=s_ref.at[pl.ds(g * CPG, CPG)],
                    dst_ref=comm_ref.at[d - 1, pl.ds(g * CPG, CPG)],
                    send_sem=send_sems.at[g, d - 1],
                    recv_sem=recv_sems.at[g, d - 1],
                    device_id=((my + d) % N_DEV,),
                    device_id_type=pl.DeviceIdType.MESH,
                )
                rdma.start()
                rdmas.append(rdma)

        k = 0
        for g in range(G):
            for _ in range(N_DEV - 1):
                rdmas[k].wait()
                k += 1
            s_tot = s_ref[pl.ds(g * CPG, CPG), :] + jnp.sum(
                comm_ref[:, pl.ds(g * CPG, CPG), :], axis=0
            )
            inv = 1.0 / s_tot
            for j in range(CPG):
                i = g * CPG + j
                eb = e_ref[pl.ds(i * CHUNK, CHUNK), :].reshape(
                    1, CHUNK, n_loc
                ).astype(jnp.float32)
                out_ref[pl.ds(i * CHUNK, CHUNK), :] = (
                    (eb * inv[j : j + 1, :][:, :, None])
                    .reshape(CHUNK, n_loc)
                    .astype(jnp.bfloat16)
                )

    return pl.pallas_call(
        body,
        out_shape=jax.ShapeDtypeStruct((m_rows, n_loc), jnp.bfloat16),
        in_specs=[pl.BlockSpec(memory_space=pltpu.VMEM)],
        out_specs=pl.BlockSpec(memory_space=pltpu.VMEM),
        scratch_shapes=[
            pltpu.VMEM((M_ROWS, n_loc), jnp.bfloat16),
            pltpu.VMEM((N_CHUNK, CHUNK), jnp.float32),
            pltpu.VMEM((N_DEV - 1, N_CHUNK, CHUNK), jnp.float32),
            pltpu.SemaphoreType.DMA((G, N_DEV - 1)),
            pltpu.SemaphoreType.DMA((G, N_DEV - 1)),
        ],
        compiler_params=pltpu.CompilerParams(collective_id=0),
    )(x)
